# baseline (speedup 1.0000x reference)
import functools

import jax
import jax.numpy as jnp
from jax import lax
from jax.experimental import pallas as pl
from jax.experimental.pallas import tpu as pltpu
from jax.experimental.pallas import tpu_sc as plsc

_NUM_EMB = 1000000
_DIM = 32
_BATCH = 16384

_NC = 2
_NS = 16
_NW = _NC * _NS
_BPW = _BATCH // _NW

_sc_mesh = plsc.VectorSubcoreMesh(core_axis_name="c", subcore_axis_name="s")


@functools.partial(
    pl.kernel,
    mesh=_sc_mesh,
    compiler_params=pltpu.CompilerParams(use_tc_tiling_on_sc=False),
    out_type=[jax.ShapeDtypeStruct((_BATCH, _DIM), jnp.bfloat16)] * 3,
    scratch_types=[
        pltpu.VMEM((_BPW,), jnp.int32),
        pltpu.VMEM((_BPW, _DIM), jnp.bfloat16),
        pltpu.VMEM((_BPW, _DIM), jnp.bfloat16),
        pltpu.VMEM((_BPW, _DIM), jnp.bfloat16),
        pltpu.SemaphoreType.DMA,
        pltpu.SemaphoreType.DMA,
        pltpu.SemaphoreType.DMA,
    ],
)
def _gather3(idx_hbm, tw_hbm, pw_hbm, vw_hbm, out_t, out_p, out_v,
             idx_v, rows_t, rows_p, rows_v, sem_t, sem_p, sem_v):
    wid = lax.axis_index("s") * _NC + lax.axis_index("c")
    base = wid * _BPW
    pltpu.sync_copy(idx_hbm.at[pl.ds(base, _BPW)], idx_v)
    ct = pltpu.async_copy(tw_hbm.at[idx_v], rows_t, sem_t)
    cp = pltpu.async_copy(pw_hbm.at[idx_v], rows_p, sem_p)
    cv = pltpu.async_copy(vw_hbm.at[idx_v], rows_v, sem_v)
    ct.wait()
    pltpu.sync_copy(rows_t, out_t.at[pl.ds(base, _BPW)])
    cp.wait()
    pltpu.sync_copy(rows_p, out_p.at[pl.ds(base, _BPW)])
    cv.wait()
    pltpu.sync_copy(rows_v, out_v.at[pl.ds(base, _BPW)])


def _trig_body(t_ref, p_ref, v_ref, ha_ref, hai_ref, hb_ref, hbi_ref):
    t = t_ref[...].astype(jnp.float32)
    p = p_ref[...].astype(jnp.float32)
    v = v_ref[...].astype(jnp.float32)
    st = jnp.sin(t)
    stsp = st * jnp.sin(p)
    ha_ref[...] = jnp.cos(t)
    hai_ref[...] = st * jnp.cos(p)
    hb_ref[...] = stsp * jnp.cos(v)
    hbi_ref[...] = stsp * jnp.sin(v)


_ROWS2D = _BATCH * _DIM // 128
_TBLK = 512


def _trig(theta, phi, varphi):
    in_spec = pl.BlockSpec((_TBLK, 128), lambda i: (i, 0))
    out_spec = pl.BlockSpec((_TBLK, 128), lambda i: (i, 0))
    out = jax.ShapeDtypeStruct((_ROWS2D, 128), jnp.float32)
    return pl.pallas_call(
        _trig_body,
        grid=(_ROWS2D // _TBLK,),
        in_specs=[in_spec, in_spec, in_spec],
        out_specs=[out_spec, out_spec, out_spec, out_spec],
        out_shape=[out, out, out, out],
    )(theta, phi, varphi)


@jax.jit
def kernel(h_idx, theta_w, phi_w, varphi_w):
    idx = h_idx.astype(jnp.int32)
    tw = theta_w.astype(jnp.bfloat16)
    pw = phi_w.astype(jnp.bfloat16)
    vw = varphi_w.astype(jnp.bfloat16)
    theta, phi, varphi = _gather3(idx, tw, pw, vw)
    theta = theta.reshape(_ROWS2D, 128)
    phi = phi.reshape(_ROWS2D, 128)
    varphi = varphi.reshape(_ROWS2D, 128)
    ha, hai, hb, hbi = _trig(theta, phi, varphi)
    shape = (_BATCH, _DIM)
    return ((ha.reshape(shape), hai.reshape(shape)),
            (hb.reshape(shape), hbi.reshape(shape)))


# speedup vs baseline: 1.1713x; 1.1713x over previous
import functools

import jax
import jax.numpy as jnp
from jax import lax
from jax.experimental import pallas as pl
from jax.experimental.pallas import tpu as pltpu
from jax.experimental.pallas import tpu_sc as plsc

_NUM_EMB = 1000000
_DIM = 32
_BATCH = 16384

_NC = 2
_NS = 16
_NW = _NC * _NS
_BPW = _BATCH // _NW

_sc_mesh = plsc.VectorSubcoreMesh(core_axis_name="c", subcore_axis_name="s")


@functools.partial(
    pl.kernel,
    mesh=_sc_mesh,
    compiler_params=pltpu.CompilerParams(use_tc_tiling_on_sc=False),
    out_type=[jax.ShapeDtypeStruct((_BATCH, _DIM), jnp.float32)] * 3,
    scratch_types=[
        pltpu.VMEM((_BPW,), jnp.int32),
        pltpu.VMEM((_BPW, _DIM), jnp.float32),
        pltpu.VMEM((_BPW, _DIM), jnp.float32),
        pltpu.VMEM((_BPW, _DIM), jnp.float32),
        pltpu.SemaphoreType.DMA,
        pltpu.SemaphoreType.DMA,
        pltpu.SemaphoreType.DMA,
    ],
)
def _gather3(idx_hbm, tw_hbm, pw_hbm, vw_hbm, out_t, out_p, out_v,
             idx_v, rows_t, rows_p, rows_v, sem_t, sem_p, sem_v):
    wid = lax.axis_index("s") * _NC + lax.axis_index("c")
    base = wid * _BPW
    pltpu.sync_copy(idx_hbm.at[pl.ds(base, _BPW)], idx_v)
    ct = pltpu.async_copy(tw_hbm.at[idx_v], rows_t, sem_t)
    cp = pltpu.async_copy(pw_hbm.at[idx_v], rows_p, sem_p)
    cv = pltpu.async_copy(vw_hbm.at[idx_v], rows_v, sem_v)
    ct.wait()
    pltpu.sync_copy(rows_t, out_t.at[pl.ds(base, _BPW)])
    cp.wait()
    pltpu.sync_copy(rows_p, out_p.at[pl.ds(base, _BPW)])
    cv.wait()
    pltpu.sync_copy(rows_v, out_v.at[pl.ds(base, _BPW)])


def _trig_body(t_ref, p_ref, v_ref, ha_ref, hai_ref, hb_ref, hbi_ref):
    t = t_ref[...]
    p = p_ref[...]
    v = v_ref[...]
    st = jnp.sin(t)
    stsp = st * jnp.sin(p)
    ha_ref[...] = jnp.cos(t)
    hai_ref[...] = st * jnp.cos(p)
    hb_ref[...] = stsp * jnp.cos(v)
    hbi_ref[...] = stsp * jnp.sin(v)


_ROWS2D = _BATCH * _DIM // 128
_TBLK = 512


def _trig(theta, phi, varphi):
    spec = pl.BlockSpec((_TBLK, 128), lambda i: (i, 0))
    out = jax.ShapeDtypeStruct((_ROWS2D, 128), jnp.float32)
    return pl.pallas_call(
        _trig_body,
        grid=(_ROWS2D // _TBLK,),
        in_specs=[spec, spec, spec],
        out_specs=[spec, spec, spec, spec],
        out_shape=[out, out, out, out],
    )(theta, phi, varphi)


@jax.jit
def kernel(h_idx, theta_w, phi_w, varphi_w):
    idx = h_idx.astype(jnp.int32)
    theta, phi, varphi = _gather3(idx, theta_w, phi_w, varphi_w)
    theta = theta.reshape(_ROWS2D, 128)
    phi = phi.reshape(_ROWS2D, 128)
    varphi = varphi.reshape(_ROWS2D, 128)
    ha, hai, hb, hbi = _trig(theta, phi, varphi)
    shape = (_BATCH, _DIM)
    return ((ha.reshape(shape), hai.reshape(shape)),
            (hb.reshape(shape), hbi.reshape(shape)))
